# Initial kernel scaffold; baseline (speedup 1.0000x reference)
import jax
import jax.numpy as jnp
from jax.experimental import pallas as pl


def kernel(x, edge_index, edge_attr, batch, W1, b1, W2, b2, W3, b3, Wl, bl):
    raise NotImplementedError("write your pallas kernel here")


# speedup vs baseline: 3.9581x; 3.9581x over previous
import functools

import jax
import jax.numpy as jnp
from jax import lax
from jax.experimental import pallas as pl
from jax.experimental.pallas import tpu as pltpu
from jax.experimental.pallas import tpu_sc as plsc

N = 10000
NP = 10240
E = 320000
G = 64
NCORES = 2
NSUB = 16
NTILES = NCORES * NSUB

_f32 = jnp.float32
_i32 = jnp.int32

_SC_MESH = plsc.VectorSubcoreMesh(
    core_axis_name="c", subcore_axis_name="s", num_cores=NCORES, num_subcores=NSUB
)
_SC_PARAMS = pltpu.CompilerParams(needs_layout_passes=False)


def _wid():
    return lax.axis_index("c") * NSUB + lax.axis_index("s")


_EPT = E // NTILES


def _deg_body(row_hbm, col_hbm, attr_hbm, out_hbm, acc_v, r_v, c_v, a_v):
    wid = _wid()
    z16 = jnp.zeros((16,), _f32)
    j0 = jnp.zeros((16,), _i32)

    @pl.loop(0, N // 16)
    def _(i):
        acc_v[0, pl.ds(i * 16, 16)] = z16

    e0 = wid * _EPT
    pltpu.sync_copy(row_hbm.at[pl.ds(e0, _EPT)], r_v)
    pltpu.sync_copy(col_hbm.at[pl.ds(e0, _EPT)], c_v)
    pltpu.sync_copy(attr_hbm.at[pl.ds(e0, _EPT)], a_v)

    @pl.loop(0, _EPT // 16)
    def _(st):
        o = st * 16
        r16 = r_v[pl.ds(o, 16)]
        c16 = c_v[pl.ds(o, 16)]
        a16 = a_v[pl.ds(o, 16)]
        w16 = jnp.where(r16 != c16, a16, 0.0)
        plsc.addupdate_scatter(acc_v, [j0, r16], w16)

    pltpu.sync_copy(acc_v, out_hbm.at[pl.ds(wid, 1)])


def _deg_call(row, col, attr):
    k = pl.kernel(
        _deg_body,
        out_type=jax.ShapeDtypeStruct((NTILES, N), _f32),
        mesh=_SC_MESH,
        scratch_types=[
            pltpu.VMEM((1, N), _f32),
            pltpu.VMEM((_EPT,), _i32),
            pltpu.VMEM((_EPT,), _i32),
            pltpu.VMEM((_EPT,), _f32),
        ],
        compiler_params=_SC_PARAMS,
    )
    return k(row, col, attr)


def _dinv_body(degp_ref, out_ref):
    deg = jnp.sum(degp_ref[...], axis=0, keepdims=True)
    out_ref[...] = jnp.where(deg > 0.0, lax.rsqrt(jnp.maximum(deg, 1e-30)), 0.0)


def _dinv_call(degp):
    return pl.pallas_call(
        _dinv_body,
        out_shape=jax.ShapeDtypeStruct((1, N), _f32),
    )(degp)


def _norm_body(row_hbm, col_hbm, attr_hbm, dinv_hbm, nm_hbm, rc_hbm,
               dinv_v, r_v, c_v, a_v, nm_v, rc_v):
    wid = _wid()
    e0 = wid * _EPT
    pltpu.sync_copy(dinv_hbm, dinv_v)
    pltpu.sync_copy(row_hbm.at[pl.ds(e0, _EPT)], r_v)
    pltpu.sync_copy(col_hbm.at[pl.ds(e0, _EPT)], c_v)
    pltpu.sync_copy(attr_hbm.at[pl.ds(e0, _EPT)], a_v)

    @pl.loop(0, _EPT // 16)
    def _(st):
        o = st * 16
        r16 = r_v[pl.ds(o, 16)]
        c16 = c_v[pl.ds(o, 16)]
        a16 = a_v[pl.ds(o, 16)]
        dr = plsc.load_gather(dinv_v, [r16])
        dc = plsc.load_gather(dinv_v, [c16])
        w16 = jnp.where(r16 != c16, a16, 0.0)
        nm_v[pl.ds(o, 16)] = -(dr * w16) * dc
        rc_v[pl.ds(o, 16)] = r16 * 16384 + c16

    pltpu.sync_copy(nm_v, nm_hbm.at[pl.ds(e0, _EPT)])
    pltpu.sync_copy(rc_v, rc_hbm.at[pl.ds(e0, _EPT)])


def _norm_call(row, col, attr, dinv):
    k = pl.kernel(
        _norm_body,
        out_type=(
            jax.ShapeDtypeStruct((E,), _f32),
            jax.ShapeDtypeStruct((E,), _i32),
        ),
        mesh=_SC_MESH,
        scratch_types=[
            pltpu.VMEM((N,), _f32),
            pltpu.VMEM((_EPT,), _i32),
            pltpu.VMEM((_EPT,), _i32),
            pltpu.VMEM((_EPT,), _f32),
            pltpu.VMEM((_EPT,), _f32),
            pltpu.VMEM((_EPT,), _i32),
        ],
        compiler_params=_SC_PARAMS,
    )
    return k(row, col, attr, dinv)


_CE = 10000


def _make_prop(F, te, fuse_relu=False):
    fpt = F // (NTILES // te)
    ne = E // te
    nch = ne // _CE

    def body(tab_hbm, rc_hbm, nm_hbm, *rest):
        if fuse_relu:
            ct_hbm, out_hbm, tab_v, acc_v, rc_v, nm_v = rest
        else:
            out_hbm, tab_v, acc_v, rc_v, nm_v = rest
        cid = lax.axis_index("c")
        sid = lax.axis_index("s")
        if te == 1:
            f0 = (cid * NSUB + sid) * fpt
            ebase = 0
            obase = f0
        else:
            f0 = sid * fpt
            ebase = cid * ne
            obase = cid * F + f0

        pltpu.sync_copy(tab_hbm.at[pl.ds(f0, fpt)], tab_v)

        z16 = jnp.zeros((16,), _f32)
        for j in range(fpt):
            @pl.loop(0, NP // 16)
            def _(i):
                acc_v[j, pl.ds(i * 16, 16)] = z16

        jf = [jnp.full((16,), j, _i32) for j in range(fpt)]

        @pl.loop(0, nch)
        def _(ch):
            e0 = ebase + ch * _CE
            pltpu.sync_copy(rc_hbm.at[pl.ds(e0, _CE)], rc_v)
            pltpu.sync_copy(nm_hbm.at[pl.ds(e0, _CE)], nm_v)

            @pl.loop(0, _CE // 16)
            def _(st):
                o = st * 16
                pk = rc_v[pl.ds(o, 16)]
                r16 = lax.shift_right_logical(pk, 14)
                c16 = lax.bitwise_and(pk, 16383)
                n16 = nm_v[pl.ds(o, 16)]
                for j in range(fpt):
                    vals = plsc.load_gather(tab_v, [jf[j], r16])
                    plsc.addupdate_scatter(acc_v, [jf[j], c16], vals * n16)

        if fuse_relu:
            pltpu.sync_copy(ct_hbm.at[pl.ds(f0, fpt)], tab_v)
            for j in range(fpt):
                @pl.loop(0, NP // 16)
                def _(i):
                    sl = pl.ds(i * 16, 16)
                    acc_v[j, sl] = jnp.maximum(tab_v[j, sl] + 2.0 * acc_v[j, sl], 0.0)

        pltpu.sync_copy(acc_v, out_hbm.at[pl.ds(obase, fpt)])

    out_rows = F if te == 1 else 2 * F
    scratch = [
        pltpu.VMEM((fpt, NP), _f32),
        pltpu.VMEM((fpt, NP), _f32),
        pltpu.VMEM((_CE,), _i32),
        pltpu.VMEM((_CE,), _f32),
    ]

    def call(tab, rc, nm, ct=None):
        k = pl.kernel(
            body,
            out_type=jax.ShapeDtypeStruct((out_rows, NP), _f32),
            mesh=_SC_MESH,
            scratch_types=scratch,
            compiler_params=_SC_PARAMS,
        )
        if fuse_relu:
            return k(tab, rc, nm, ct)
        return k(tab, rc, nm)

    return call


_prop128 = _make_prop(128, 1)
_prop128_relu = _make_prop(128, 1, fuse_relu=True)
_prop64 = _make_prop(64, 2)
_prop32 = _make_prop(32, 2)


def _mm_a_body(h_ref, w_ref, b_ref, out_ref):
    out_ref[...] = (
        jnp.dot(w_ref[...], h_ref[...], preferred_element_type=_f32) + b_ref[...]
    )


def _mm_a(h, w0pt, b):
    fo = w0pt.shape[0]
    return pl.pallas_call(
        _mm_a_body,
        out_shape=jax.ShapeDtypeStruct((fo, NP), _f32),
    )(h, w0pt, b.reshape(fo, 1))


def _make_mm_b(partials):
    def body(t1_ref, w1t_ref, w2t_ref, ct_ref, z_ref, cout_ref):
        if partials:
            fi = t1_ref.shape[0] // 2
            t1 = t1_ref[pl.ds(0, fi), :] + t1_ref[pl.ds(fi, fi), :]
        else:
            t1 = t1_ref[...]
        z_ref[...] = jnp.dot(w2t_ref[...], t1, preferred_element_type=_f32)
        cout_ref[...] = ct_ref[...] + jnp.dot(
            w1t_ref[...], t1, preferred_element_type=_f32
        )

    def call(t1, w1t, w2t, ct):
        fo = w1t.shape[0]
        return pl.pallas_call(
            body,
            out_shape=(
                jax.ShapeDtypeStruct((fo, NP), _f32),
                jax.ShapeDtypeStruct((fo, NP), _f32),
            ),
        )(t1, w1t, w2t, ct)

    return call


_mm_b = _make_mm_b(False)
_mm_b_p = _make_mm_b(True)


def _relu_mm_a_body(c_ref, yp_ref, w_ref, b_ref, h_ref, cout_ref):
    fi = c_ref.shape[0]
    h = jnp.maximum(c_ref[...] + 2.0 * (yp_ref[pl.ds(0, fi), :] + yp_ref[pl.ds(fi, fi), :]), 0.0)
    h_ref[...] = h
    cout_ref[...] = jnp.dot(w_ref[...], h, preferred_element_type=_f32) + b_ref[...]


def _relu_mm_a(c, yp, w0pt, b):
    fi, fo = w0pt.shape[1], w0pt.shape[0]
    return pl.pallas_call(
        _relu_mm_a_body,
        out_shape=(
            jax.ShapeDtypeStruct((fi, NP), _f32),
            jax.ShapeDtypeStruct((fo, NP), _f32),
        ),
    )(c, yp, w0pt, b.reshape(fo, 1))


def _final_body(c_ref, yp_ref, batch_ref, wl_ref, bl_ref, out_ref):
    fi = c_ref.shape[0]
    h = jnp.maximum(c_ref[...] + 2.0 * (yp_ref[pl.ds(0, fi), :] + yp_ref[pl.ds(fi, fi), :]), 0.0)
    grp = batch_ref[...]
    iot = lax.broadcasted_iota(_i32, (G, NP), 0)
    m = (grp == iot).astype(_f32)
    sums = lax.dot_general(m, h, (((1,), (1,)), ((), ())), preferred_element_type=_f32)
    cnt = jnp.sum(m, axis=1, keepdims=True)
    pooled = sums / jnp.maximum(cnt, 1.0)
    out_ref[...] = jnp.dot(pooled, wl_ref[...], preferred_element_type=_f32) + bl_ref[...]


def _final_call(c, yp, batch_p, wl, bl):
    return pl.pallas_call(
        _final_body,
        out_shape=jax.ShapeDtypeStruct((G, 2), _f32),
    )(c, yp, batch_p, wl, bl.reshape(1, 2))


@jax.jit
def kernel(x, edge_index, edge_attr, batch, W1, b1, W2, b2, W3, b3, Wl, bl):
    row = edge_index[0]
    col = edge_index[1]
    xt = jnp.pad(x.T, ((0, 0), (0, NP - N)))
    batch_p = jnp.pad(batch.astype(_i32), (0, NP - N), constant_values=G).reshape(1, NP)

    w0pt1 = (W1[0] - W1[2]).T
    w0pt2 = (W2[0] - W2[2]).T
    w0pt3 = (W3[0] - W3[2]).T

    degp = _deg_call(row, col, edge_attr)
    dinv = _dinv_call(degp).reshape(N)
    nm, rc = _norm_call(row, col, edge_attr, dinv)

    c1 = _mm_a(xt, w0pt1, b1)
    t1 = _prop128(xt, rc, nm)
    z1, c1b = _mm_b(t1, W1[1].T, W1[2].T, c1)
    h2 = _prop128_relu(z1, rc, nm, ct=c1b)

    c2 = _mm_a(h2, w0pt2, b2)
    t2 = _prop128(h2, rc, nm)
    z2, c2b = _mm_b(t2, W2[1].T, W2[2].T, c2)
    y2p = _prop64(z2, rc, nm)

    h3, c3 = _relu_mm_a(c2b, y2p, w0pt3, b3)
    t3p = _prop64(h3, rc, nm)
    z3, c3b = _mm_b_p(t3p, W3[1].T, W3[2].T, c3)
    y3p = _prop32(z3, rc, nm)

    return _final_call(c3b, y3p, batch_p, Wl, bl)


# speedup vs baseline: 4.5400x; 1.1470x over previous
import functools

import jax
import jax.numpy as jnp
from jax import lax
from jax.experimental import pallas as pl
from jax.experimental.pallas import tpu as pltpu
from jax.experimental.pallas import tpu_sc as plsc

N = 10000
NP = 10240
E = 320000
G = 64
NCORES = 2
NSUB = 16
NTILES = NCORES * NSUB

_f32 = jnp.float32
_i32 = jnp.int32

_SC_MESH = plsc.VectorSubcoreMesh(
    core_axis_name="c", subcore_axis_name="s", num_cores=NCORES, num_subcores=NSUB
)
_SC_PARAMS = pltpu.CompilerParams(needs_layout_passes=False)


def _wid():
    return lax.axis_index("c") * NSUB + lax.axis_index("s")


_EPT = E // NTILES


def _deg_body(row_hbm, col_hbm, attr_hbm, out_hbm, acc_v, r_v, c_v, a_v):
    wid = _wid()
    z16 = jnp.zeros((16,), _f32)
    j0 = jnp.zeros((16,), _i32)

    @pl.loop(0, N // 16, unroll=_UNROLL)
    def _(i):
        acc_v[0, pl.ds(i * 16, 16)] = z16

    e0 = wid * _EPT
    pltpu.sync_copy(row_hbm.at[pl.ds(e0, _EPT)], r_v)
    pltpu.sync_copy(col_hbm.at[pl.ds(e0, _EPT)], c_v)
    pltpu.sync_copy(attr_hbm.at[pl.ds(e0, _EPT)], a_v)

    @pl.loop(0, _EPT // 16, unroll=_UNROLL)
    def _(st):
        o = st * 16
        r16 = r_v[pl.ds(o, 16)]
        c16 = c_v[pl.ds(o, 16)]
        a16 = a_v[pl.ds(o, 16)]
        w16 = jnp.where(r16 != c16, a16, 0.0)
        plsc.addupdate_scatter(acc_v, [j0, r16], w16)

    pltpu.sync_copy(acc_v, out_hbm.at[pl.ds(wid, 1)])


def _deg_call(row, col, attr):
    k = pl.kernel(
        _deg_body,
        out_type=jax.ShapeDtypeStruct((NTILES, N), _f32),
        mesh=_SC_MESH,
        scratch_types=[
            pltpu.VMEM((1, N), _f32),
            pltpu.VMEM((_EPT,), _i32),
            pltpu.VMEM((_EPT,), _i32),
            pltpu.VMEM((_EPT,), _f32),
        ],
        compiler_params=_SC_PARAMS,
    )
    return k(row, col, attr)


def _dinv_body(degp_ref, out_ref):
    deg = jnp.sum(degp_ref[...], axis=0, keepdims=True)
    out_ref[...] = jnp.where(deg > 0.0, lax.rsqrt(jnp.maximum(deg, 1e-30)), 0.0)


def _dinv_call(degp):
    return pl.pallas_call(
        _dinv_body,
        out_shape=jax.ShapeDtypeStruct((1, N), _f32),
    )(degp)


def _norm_body(row_hbm, col_hbm, attr_hbm, dinv_hbm, nm_hbm, rc_hbm,
               dinv_v, r_v, c_v, a_v, nm_v, rc_v):
    wid = _wid()
    e0 = wid * _EPT
    pltpu.sync_copy(dinv_hbm, dinv_v)
    pltpu.sync_copy(row_hbm.at[pl.ds(e0, _EPT)], r_v)
    pltpu.sync_copy(col_hbm.at[pl.ds(e0, _EPT)], c_v)
    pltpu.sync_copy(attr_hbm.at[pl.ds(e0, _EPT)], a_v)

    @pl.loop(0, _EPT // 16, unroll=_UNROLL)
    def _(st):
        o = st * 16
        r16 = r_v[pl.ds(o, 16)]
        c16 = c_v[pl.ds(o, 16)]
        a16 = a_v[pl.ds(o, 16)]
        dr = plsc.load_gather(dinv_v, [r16])
        dc = plsc.load_gather(dinv_v, [c16])
        w16 = jnp.where(r16 != c16, a16, 0.0)
        nm_v[pl.ds(o, 16)] = -(dr * w16) * dc
        rc_v[pl.ds(o, 16)] = r16 * 16384 + c16

    pltpu.sync_copy(nm_v, nm_hbm.at[pl.ds(e0, _EPT)])
    pltpu.sync_copy(rc_v, rc_hbm.at[pl.ds(e0, _EPT)])


def _norm_call(row, col, attr, dinv):
    k = pl.kernel(
        _norm_body,
        out_type=(
            jax.ShapeDtypeStruct((E,), _f32),
            jax.ShapeDtypeStruct((E,), _i32),
        ),
        mesh=_SC_MESH,
        scratch_types=[
            pltpu.VMEM((N,), _f32),
            pltpu.VMEM((_EPT,), _i32),
            pltpu.VMEM((_EPT,), _i32),
            pltpu.VMEM((_EPT,), _f32),
            pltpu.VMEM((_EPT,), _f32),
            pltpu.VMEM((_EPT,), _i32),
        ],
        compiler_params=_SC_PARAMS,
    )
    return k(row, col, attr, dinv)


_CE = 8000
_UNROLL = 8


def _make_prop(F, te, fuse_relu=False):
    fpt = F // (NTILES // te)
    ne = E // te
    nch = ne // _CE
    assert nch % 2 == 0

    def body(tab_hbm, rc_hbm, nm_hbm, *rest):
        if fuse_relu:
            ct_hbm, out_hbm, tab_v, acc_v, rc0_v, rc1_v, nm0_v, nm1_v, sem0, sem1 = rest
        else:
            out_hbm, tab_v, acc_v, rc0_v, rc1_v, nm0_v, nm1_v, sem0, sem1 = rest
        rcb = (rc0_v, rc1_v)
        nmb = (nm0_v, nm1_v)
        sems = (sem0, sem1)
        cid = lax.axis_index("c")
        sid = lax.axis_index("s")
        if te == 1:
            f0 = (cid * NSUB + sid) * fpt
            ebase = 0
            obase = f0
        else:
            f0 = sid * fpt
            ebase = cid * ne
            obase = cid * F + f0

        def fire(ch, b):
            e0 = ebase + ch * _CE
            pltpu.async_copy(rc_hbm.at[pl.ds(e0, _CE)], rcb[b], sems[b])
            pltpu.async_copy(nm_hbm.at[pl.ds(e0, _CE)], nmb[b], sems[b])

        def drain(b):
            pltpu.make_async_copy(rc_hbm.at[pl.ds(0, _CE)], rcb[b], sems[b]).wait()
            pltpu.make_async_copy(nm_hbm.at[pl.ds(0, _CE)], nmb[b], sems[b]).wait()

        jf = [jnp.full((16,), j, _i32) for j in range(fpt)]

        def compute(b):
            @pl.loop(0, _CE // 16, unroll=_UNROLL)
            def _(st):
                o = st * 16
                pk = rcb[b][pl.ds(o, 16)]
                r16 = lax.shift_right_logical(pk, 14)
                c16 = lax.bitwise_and(pk, 16383)
                n16 = nmb[b][pl.ds(o, 16)]
                for j in range(fpt):
                    vals = plsc.load_gather(tab_v, [jf[j], r16])
                    plsc.addupdate_scatter(acc_v, [jf[j], c16], vals * n16)

        fire(0, 0)
        pltpu.sync_copy(tab_hbm.at[pl.ds(f0, fpt)], tab_v)

        z16 = jnp.zeros((16,), _f32)
        for j in range(fpt):
            @pl.loop(0, NP // 16, unroll=_UNROLL)
            def _(i):
                acc_v[j, pl.ds(i * 16, 16)] = z16

        @pl.loop(0, nch, step=2)
        def _(ch):
            fire(ch + 1, 1)
            drain(0)
            compute(0)

            @pl.when(ch + 2 < nch)
            def _():
                fire(ch + 2, 0)

            drain(1)
            compute(1)

        if fuse_relu:
            pltpu.sync_copy(ct_hbm.at[pl.ds(f0, fpt)], tab_v)
            for j in range(fpt):
                @pl.loop(0, NP // 16, unroll=_UNROLL)
                def _(i):
                    sl = pl.ds(i * 16, 16)
                    acc_v[j, sl] = jnp.maximum(tab_v[j, sl] + 2.0 * acc_v[j, sl], 0.0)

        pltpu.sync_copy(acc_v, out_hbm.at[pl.ds(obase, fpt)])

    out_rows = F if te == 1 else 2 * F
    scratch = [
        pltpu.VMEM((fpt, NP), _f32),
        pltpu.VMEM((fpt, NP), _f32),
        pltpu.VMEM((_CE,), _i32),
        pltpu.VMEM((_CE,), _i32),
        pltpu.VMEM((_CE,), _f32),
        pltpu.VMEM((_CE,), _f32),
        pltpu.SemaphoreType.DMA,
        pltpu.SemaphoreType.DMA,
    ]

    def call(tab, rc, nm, ct=None):
        k = pl.kernel(
            body,
            out_type=jax.ShapeDtypeStruct((out_rows, NP), _f32),
            mesh=_SC_MESH,
            scratch_types=scratch,
            compiler_params=_SC_PARAMS,
        )
        if fuse_relu:
            return k(tab, rc, nm, ct)
        return k(tab, rc, nm)

    return call


_prop128 = _make_prop(128, 1)
_prop128_relu = _make_prop(128, 1, fuse_relu=True)
_prop64 = _make_prop(64, 2)
_prop32 = _make_prop(32, 2)


def _mm_a_body(h_ref, w_ref, b_ref, out_ref):
    out_ref[...] = (
        jnp.dot(w_ref[...], h_ref[...], preferred_element_type=_f32) + b_ref[...]
    )


def _mm_a(h, w0pt, b):
    fo = w0pt.shape[0]
    return pl.pallas_call(
        _mm_a_body,
        out_shape=jax.ShapeDtypeStruct((fo, NP), _f32),
    )(h, w0pt, b.reshape(fo, 1))


def _make_mm_b(partials):
    def body(t1_ref, w1t_ref, w2t_ref, ct_ref, z_ref, cout_ref):
        if partials:
            fi = t1_ref.shape[0] // 2
            t1 = t1_ref[pl.ds(0, fi), :] + t1_ref[pl.ds(fi, fi), :]
        else:
            t1 = t1_ref[...]
        z_ref[...] = jnp.dot(w2t_ref[...], t1, preferred_element_type=_f32)
        cout_ref[...] = ct_ref[...] + jnp.dot(
            w1t_ref[...], t1, preferred_element_type=_f32
        )

    def call(t1, w1t, w2t, ct):
        fo = w1t.shape[0]
        return pl.pallas_call(
            body,
            out_shape=(
                jax.ShapeDtypeStruct((fo, NP), _f32),
                jax.ShapeDtypeStruct((fo, NP), _f32),
            ),
        )(t1, w1t, w2t, ct)

    return call


_mm_b = _make_mm_b(False)
_mm_b_p = _make_mm_b(True)


def _relu_mm_a_body(c_ref, yp_ref, w_ref, b_ref, h_ref, cout_ref):
    fi = c_ref.shape[0]
    h = jnp.maximum(c_ref[...] + 2.0 * (yp_ref[pl.ds(0, fi), :] + yp_ref[pl.ds(fi, fi), :]), 0.0)
    h_ref[...] = h
    cout_ref[...] = jnp.dot(w_ref[...], h, preferred_element_type=_f32) + b_ref[...]


def _relu_mm_a(c, yp, w0pt, b):
    fi, fo = w0pt.shape[1], w0pt.shape[0]
    return pl.pallas_call(
        _relu_mm_a_body,
        out_shape=(
            jax.ShapeDtypeStruct((fi, NP), _f32),
            jax.ShapeDtypeStruct((fo, NP), _f32),
        ),
    )(c, yp, w0pt, b.reshape(fo, 1))


def _final_body(c_ref, yp_ref, batch_ref, wl_ref, bl_ref, out_ref):
    fi = c_ref.shape[0]
    h = jnp.maximum(c_ref[...] + 2.0 * (yp_ref[pl.ds(0, fi), :] + yp_ref[pl.ds(fi, fi), :]), 0.0)
    grp = batch_ref[...]
    iot = lax.broadcasted_iota(_i32, (G, NP), 0)
    m = (grp == iot).astype(_f32)
    sums = lax.dot_general(m, h, (((1,), (1,)), ((), ())), preferred_element_type=_f32)
    cnt = jnp.sum(m, axis=1, keepdims=True)
    pooled = sums / jnp.maximum(cnt, 1.0)
    out_ref[...] = jnp.dot(pooled, wl_ref[...], preferred_element_type=_f32) + bl_ref[...]


def _final_call(c, yp, batch_p, wl, bl):
    return pl.pallas_call(
        _final_body,
        out_shape=jax.ShapeDtypeStruct((G, 2), _f32),
    )(c, yp, batch_p, wl, bl.reshape(1, 2))


@jax.jit
def kernel(x, edge_index, edge_attr, batch, W1, b1, W2, b2, W3, b3, Wl, bl):
    row = edge_index[0]
    col = edge_index[1]
    xt = jnp.pad(x.T, ((0, 0), (0, NP - N)))
    batch_p = jnp.pad(batch.astype(_i32), (0, NP - N), constant_values=G).reshape(1, NP)

    w0pt1 = (W1[0] - W1[2]).T
    w0pt2 = (W2[0] - W2[2]).T
    w0pt3 = (W3[0] - W3[2]).T

    degp = _deg_call(row, col, edge_attr)
    dinv = _dinv_call(degp).reshape(N)
    nm, rc = _norm_call(row, col, edge_attr, dinv)

    c1 = _mm_a(xt, w0pt1, b1)
    t1 = _prop128(xt, rc, nm)
    z1, c1b = _mm_b(t1, W1[1].T, W1[2].T, c1)
    h2 = _prop128_relu(z1, rc, nm, ct=c1b)

    c2 = _mm_a(h2, w0pt2, b2)
    t2 = _prop128(h2, rc, nm)
    z2, c2b = _mm_b(t2, W2[1].T, W2[2].T, c2)
    y2p = _prop64(z2, rc, nm)

    h3, c3 = _relu_mm_a(c2b, y2p, w0pt3, b3)
    t3p = _prop64(h3, rc, nm)
    z3, c3b = _mm_b_p(t3p, W3[1].T, W3[2].T, c3)
    y3p = _prop32(z3, rc, nm)

    return _final_call(c3b, y3p, batch_p, Wl, bl)


# speedup vs baseline: 12.2760x; 2.7040x over previous
import functools

import jax
import jax.numpy as jnp
from jax import lax
from jax.experimental import pallas as pl
from jax.experimental.pallas import tpu as pltpu
from jax.experimental.pallas import tpu_sc as plsc

N = 10000
NP = 10240
E = 320000
G = 64
NCORES = 2
NSUB = 16
NTILES = NCORES * NSUB

_f32 = jnp.float32
_i32 = jnp.int32

_SC_MESH = plsc.VectorSubcoreMesh(
    core_axis_name="c", subcore_axis_name="s", num_cores=NCORES, num_subcores=NSUB
)
_SC_PARAMS = pltpu.CompilerParams(needs_layout_passes=False)


def _wid():
    return lax.axis_index("c") * NSUB + lax.axis_index("s")


_EPT = E // NTILES


def _deg_body(row_hbm, col_hbm, attr_hbm, out_hbm, acc_v, r_v, c_v, a_v):
    wid = _wid()
    z16 = jnp.zeros((16,), _f32)
    j0 = jnp.zeros((16,), _i32)

    @pl.loop(0, N // 16, unroll=_UNROLL)
    def _(i):
        acc_v[0, pl.ds(i * 16, 16)] = z16

    e0 = wid * _EPT
    pltpu.sync_copy(row_hbm.at[pl.ds(e0, _EPT)], r_v)
    pltpu.sync_copy(col_hbm.at[pl.ds(e0, _EPT)], c_v)
    pltpu.sync_copy(attr_hbm.at[pl.ds(e0, _EPT)], a_v)

    @pl.loop(0, _EPT // 16, unroll=_UNROLL)
    def _(st):
        o = st * 16
        r16 = r_v[pl.ds(o, 16)]
        c16 = c_v[pl.ds(o, 16)]
        a16 = a_v[pl.ds(o, 16)]
        w16 = jnp.where(r16 != c16, a16, 0.0)
        plsc.addupdate_scatter(acc_v, [j0, r16], w16)

    pltpu.sync_copy(acc_v, out_hbm.at[pl.ds(wid, 1)])


def _deg_call(row, col, attr):
    k = pl.kernel(
        _deg_body,
        out_type=jax.ShapeDtypeStruct((NTILES, N), _f32),
        mesh=_SC_MESH,
        scratch_types=[
            pltpu.VMEM((1, N), _f32),
            pltpu.VMEM((_EPT,), _i32),
            pltpu.VMEM((_EPT,), _i32),
            pltpu.VMEM((_EPT,), _f32),
        ],
        compiler_params=_SC_PARAMS,
    )
    return k(row, col, attr)


def _dinv_body(degp_ref, out_ref):
    deg = jnp.sum(degp_ref[...], axis=0, keepdims=True)
    out_ref[...] = jnp.where(deg > 0.0, lax.rsqrt(jnp.maximum(deg, 1e-30)), 0.0)


def _dinv_call(degp):
    return pl.pallas_call(
        _dinv_body,
        out_shape=jax.ShapeDtypeStruct((1, N), _f32),
    )(degp)


def _norm_body(row_hbm, col_hbm, attr_hbm, dinv_hbm, nm_hbm, rc_hbm,
               dinv_v, r_v, c_v, a_v, nm_v, rc_v):
    wid = _wid()
    e0 = wid * _EPT
    pltpu.sync_copy(dinv_hbm, dinv_v)
    pltpu.sync_copy(row_hbm.at[pl.ds(e0, _EPT)], r_v)
    pltpu.sync_copy(col_hbm.at[pl.ds(e0, _EPT)], c_v)
    pltpu.sync_copy(attr_hbm.at[pl.ds(e0, _EPT)], a_v)

    @pl.loop(0, _EPT // 16, unroll=_UNROLL)
    def _(st):
        o = st * 16
        r16 = r_v[pl.ds(o, 16)]
        c16 = c_v[pl.ds(o, 16)]
        a16 = a_v[pl.ds(o, 16)]
        dr = plsc.load_gather(dinv_v, [r16])
        dc = plsc.load_gather(dinv_v, [c16])
        w16 = jnp.where(r16 != c16, a16, 0.0)
        nm_v[pl.ds(o, 16)] = -(dr * w16) * dc
        rc_v[pl.ds(o, 16)] = r16 * 16384 + c16

    pltpu.sync_copy(nm_v, nm_hbm.at[pl.ds(e0, _EPT)])
    pltpu.sync_copy(rc_v, rc_hbm.at[pl.ds(e0, _EPT)])


def _norm_call(row, col, attr, dinv):
    k = pl.kernel(
        _norm_body,
        out_type=(
            jax.ShapeDtypeStruct((E,), _f32),
            jax.ShapeDtypeStruct((E,), _i32),
        ),
        mesh=_SC_MESH,
        scratch_types=[
            pltpu.VMEM((N,), _f32),
            pltpu.VMEM((_EPT,), _i32),
            pltpu.VMEM((_EPT,), _i32),
            pltpu.VMEM((_EPT,), _f32),
            pltpu.VMEM((_EPT,), _f32),
            pltpu.VMEM((_EPT,), _i32),
        ],
        compiler_params=_SC_PARAMS,
    )
    return k(row, col, attr, dinv)


_CE = 8000
_UNROLL = 8


def _make_prop(F, te, fuse_relu=False):
    fpt = F // (NTILES // te)
    ne = E // te
    nch = ne // _CE
    assert nch % 2 == 0

    def body(tab_hbm, rc_hbm, nm_hbm, *rest):
        if fuse_relu:
            ct_hbm, out_hbm, tab_v, acc_v, rc0_v, rc1_v, nm0_v, nm1_v, sem0, sem1 = rest
        else:
            out_hbm, tab_v, acc_v, rc0_v, rc1_v, nm0_v, nm1_v, sem0, sem1 = rest
        rcb = (rc0_v, rc1_v)
        nmb = (nm0_v, nm1_v)
        sems = (sem0, sem1)
        cid = lax.axis_index("c")
        sid = lax.axis_index("s")
        if te == 1:
            f0 = (cid * NSUB + sid) * fpt
            ebase = 0
            obase = f0
        else:
            f0 = sid * fpt
            ebase = cid * ne
            obase = cid * F + f0

        def fire(ch, b):
            e0 = ebase + ch * _CE
            pltpu.async_copy(rc_hbm.at[pl.ds(e0, _CE)], rcb[b], sems[b])
            pltpu.async_copy(nm_hbm.at[pl.ds(e0, _CE)], nmb[b], sems[b])

        def drain(b):
            pltpu.make_async_copy(rc_hbm.at[pl.ds(0, _CE)], rcb[b], sems[b]).wait()
            pltpu.make_async_copy(nm_hbm.at[pl.ds(0, _CE)], nmb[b], sems[b]).wait()

        jf = [jnp.full((16,), j, _i32) for j in range(fpt)]

        def compute(b):
            @plsc.parallel_loop(0, _CE // 16, unroll=_UNROLL)
            def _(st):
                o = st * 16
                pk = rcb[b][pl.ds(o, 16)]
                r16 = lax.shift_right_logical(pk, 14)
                c16 = lax.bitwise_and(pk, 16383)
                n16 = nmb[b][pl.ds(o, 16)]
                vals = [plsc.load_gather(tab_v, [jf[j], r16]) for j in range(fpt)]
                for j in range(fpt):
                    plsc.addupdate_scatter(acc_v, [jf[j], c16], vals[j] * n16)

        fire(0, 0)
        pltpu.sync_copy(tab_hbm.at[pl.ds(f0, fpt)], tab_v)

        z16 = jnp.zeros((16,), _f32)
        for j in range(fpt):
            @pl.loop(0, NP // 16, unroll=_UNROLL)
            def _(i):
                acc_v[j, pl.ds(i * 16, 16)] = z16

        @pl.loop(0, nch, step=2)
        def _(ch):
            fire(ch + 1, 1)
            drain(0)
            compute(0)

            @pl.when(ch + 2 < nch)
            def _():
                fire(ch + 2, 0)

            drain(1)
            compute(1)

        if fuse_relu:
            pltpu.sync_copy(ct_hbm.at[pl.ds(f0, fpt)], tab_v)
            for j in range(fpt):
                @pl.loop(0, NP // 16, unroll=_UNROLL)
                def _(i):
                    sl = pl.ds(i * 16, 16)
                    acc_v[j, sl] = jnp.maximum(tab_v[j, sl] + 2.0 * acc_v[j, sl], 0.0)

        pltpu.sync_copy(acc_v, out_hbm.at[pl.ds(obase, fpt)])

    out_rows = F if te == 1 else 2 * F
    scratch = [
        pltpu.VMEM((fpt, NP), _f32),
        pltpu.VMEM((fpt, NP), _f32),
        pltpu.VMEM((_CE,), _i32),
        pltpu.VMEM((_CE,), _i32),
        pltpu.VMEM((_CE,), _f32),
        pltpu.VMEM((_CE,), _f32),
        pltpu.SemaphoreType.DMA,
        pltpu.SemaphoreType.DMA,
    ]

    def call(tab, rc, nm, ct=None):
        k = pl.kernel(
            body,
            out_type=jax.ShapeDtypeStruct((out_rows, NP), _f32),
            mesh=_SC_MESH,
            scratch_types=scratch,
            compiler_params=_SC_PARAMS,
        )
        if fuse_relu:
            return k(tab, rc, nm, ct)
        return k(tab, rc, nm)

    return call


_prop128 = _make_prop(128, 1)
_prop128_relu = _make_prop(128, 1, fuse_relu=True)
_prop64 = _make_prop(64, 2)
_prop32 = _make_prop(32, 2)


def _mm_a_body(h_ref, w_ref, b_ref, out_ref):
    out_ref[...] = (
        jnp.dot(w_ref[...], h_ref[...], preferred_element_type=_f32) + b_ref[...]
    )


def _mm_a(h, w0pt, b):
    fo = w0pt.shape[0]
    return pl.pallas_call(
        _mm_a_body,
        out_shape=jax.ShapeDtypeStruct((fo, NP), _f32),
    )(h, w0pt, b.reshape(fo, 1))


def _make_mm_b(partials):
    def body(t1_ref, w1t_ref, w2t_ref, ct_ref, z_ref, cout_ref):
        if partials:
            fi = t1_ref.shape[0] // 2
            t1 = t1_ref[pl.ds(0, fi), :] + t1_ref[pl.ds(fi, fi), :]
        else:
            t1 = t1_ref[...]
        z_ref[...] = jnp.dot(w2t_ref[...], t1, preferred_element_type=_f32)
        cout_ref[...] = ct_ref[...] + jnp.dot(
            w1t_ref[...], t1, preferred_element_type=_f32
        )

    def call(t1, w1t, w2t, ct):
        fo = w1t.shape[0]
        return pl.pallas_call(
            body,
            out_shape=(
                jax.ShapeDtypeStruct((fo, NP), _f32),
                jax.ShapeDtypeStruct((fo, NP), _f32),
            ),
        )(t1, w1t, w2t, ct)

    return call


_mm_b = _make_mm_b(False)
_mm_b_p = _make_mm_b(True)


def _relu_mm_a_body(c_ref, yp_ref, w_ref, b_ref, h_ref, cout_ref):
    fi = c_ref.shape[0]
    h = jnp.maximum(c_ref[...] + 2.0 * (yp_ref[pl.ds(0, fi), :] + yp_ref[pl.ds(fi, fi), :]), 0.0)
    h_ref[...] = h
    cout_ref[...] = jnp.dot(w_ref[...], h, preferred_element_type=_f32) + b_ref[...]


def _relu_mm_a(c, yp, w0pt, b):
    fi, fo = w0pt.shape[1], w0pt.shape[0]
    return pl.pallas_call(
        _relu_mm_a_body,
        out_shape=(
            jax.ShapeDtypeStruct((fi, NP), _f32),
            jax.ShapeDtypeStruct((fo, NP), _f32),
        ),
    )(c, yp, w0pt, b.reshape(fo, 1))


def _final_body(c_ref, yp_ref, batch_ref, wl_ref, bl_ref, out_ref):
    fi = c_ref.shape[0]
    h = jnp.maximum(c_ref[...] + 2.0 * (yp_ref[pl.ds(0, fi), :] + yp_ref[pl.ds(fi, fi), :]), 0.0)
    grp = batch_ref[...]
    iot = lax.broadcasted_iota(_i32, (G, NP), 0)
    m = (grp == iot).astype(_f32)
    sums = lax.dot_general(m, h, (((1,), (1,)), ((), ())), preferred_element_type=_f32)
    cnt = jnp.sum(m, axis=1, keepdims=True)
    pooled = sums / jnp.maximum(cnt, 1.0)
    out_ref[...] = jnp.dot(pooled, wl_ref[...], preferred_element_type=_f32) + bl_ref[...]


def _final_call(c, yp, batch_p, wl, bl):
    return pl.pallas_call(
        _final_body,
        out_shape=jax.ShapeDtypeStruct((G, 2), _f32),
    )(c, yp, batch_p, wl, bl.reshape(1, 2))


@jax.jit
def kernel(x, edge_index, edge_attr, batch, W1, b1, W2, b2, W3, b3, Wl, bl):
    row = edge_index[0]
    col = edge_index[1]
    xt = jnp.pad(x.T, ((0, 0), (0, NP - N)))
    batch_p = jnp.pad(batch.astype(_i32), (0, NP - N), constant_values=G).reshape(1, NP)

    w0pt1 = (W1[0] - W1[2]).T
    w0pt2 = (W2[0] - W2[2]).T
    w0pt3 = (W3[0] - W3[2]).T

    degp = _deg_call(row, col, edge_attr)
    dinv = _dinv_call(degp).reshape(N)
    nm, rc = _norm_call(row, col, edge_attr, dinv)

    c1 = _mm_a(xt, w0pt1, b1)
    t1 = _prop128(xt, rc, nm)
    z1, c1b = _mm_b(t1, W1[1].T, W1[2].T, c1)
    h2 = _prop128_relu(z1, rc, nm, ct=c1b)

    c2 = _mm_a(h2, w0pt2, b2)
    t2 = _prop128(h2, rc, nm)
    z2, c2b = _mm_b(t2, W2[1].T, W2[2].T, c2)
    y2p = _prop64(z2, rc, nm)

    h3, c3 = _relu_mm_a(c2b, y2p, w0pt3, b3)
    t3p = _prop64(h3, rc, nm)
    z3, c3b = _mm_b_p(t3p, W3[1].T, W3[2].T, c3)
    y3p = _prop32(z3, rc, nm)

    return _final_call(c3b, y3p, batch_p, Wl, bl)
